# baseline (speedup 1.0000x reference)
import functools

import jax
import jax.numpy as jnp
from jax.experimental import pallas as pl
from jax.experimental.pallas import tpu as pltpu


def _conv_body(x_ref, w_ref, b_ref, o_ref, scr, *, TH, W, C_in, NT, IM):
    TS = TH * W
    HW = NT * TS
    FLAT = scr.shape[2]
    col = jax.lax.broadcasted_iota(jnp.int32, (C_in, HW), 1) % W
    b = b_ref[...]
    w = w_ref[...]
    for i in range(IM):
        flat = x_ref[i].astype(jnp.bfloat16).reshape(C_in, HW)
        zf = jnp.zeros((C_in, 2 * W), jnp.bfloat16)
        zb = jnp.zeros((C_in, 4 * W), jnp.bfloat16)
        for kw in range(3):
            scr[kw, :, 0:2 * W] = zf
            scr[kw, :, FLAT - 4 * W:FLAT] = zb
        scr[0, :, W + 1:W + 1 + HW] = jnp.where(col == W - 1, 0, flat)
        scr[1, :, W:W + HW] = flat
        scr[2, :, W - 1:W - 1 + HW] = jnp.where(col == 0, 0, flat)
        for t in range(NT):
            taps = []
            for kh in range(3):
                for kw in range(3):
                    off = t * TS + kh * W
                    taps.append(scr[kw, :, off:off + TS])
            xmat = jnp.concatenate(taps, axis=0)
            acc = jnp.dot(w, xmat, preferred_element_type=jnp.float32)
            o_ref[i, :, t * TS:(t + 1) * TS] = (acc + b).astype(o_ref.dtype)


def kernel(x, weight, bias):
    N, C_in, H, W = x.shape
    C_out = weight.shape[0]
    TH = 8
    HW = H * W
    TS = TH * W
    flat = (H // TH - 1) * TS + ((TS + 2 * W + 2 + 127) // 128) * 128


    w_mat = weight.transpose(0, 2, 3, 1).reshape(C_out, 9 * C_in)
    w_mat = w_mat.astype(jnp.bfloat16)
    b_col = bias.astype(jnp.float32).reshape(C_out, 1)

    IM = 8 if N % 8 == 0 else 1
    out = pl.pallas_call(
        functools.partial(_conv_body, TH=TH, W=W, C_in=C_in,
                          NT=H // TH, IM=IM),
        out_shape=jax.ShapeDtypeStruct((N, C_out, HW), jnp.bfloat16),
        grid=(N // IM,),
        in_specs=[
            pl.BlockSpec((IM, C_in, H, W), lambda n: (n, 0, 0, 0)),
            pl.BlockSpec((C_out, 9 * C_in), lambda n: (0, 0)),
            pl.BlockSpec((C_out, 1), lambda n: (0, 0)),
        ],
        out_specs=pl.BlockSpec((IM, C_out, HW), lambda n: (n, 0, 0)),
        scratch_shapes=[pltpu.VMEM((3, C_in, flat), jnp.bfloat16)],
        compiler_params=pltpu.CompilerParams(
            dimension_semantics=("arbitrary",)),
    )(x, w_mat, b_col)
    return out.reshape(N, C_out, H, W).astype(jnp.float32)


# speedup vs baseline: 1.0037x; 1.0037x over previous
import functools

import jax
import jax.numpy as jnp
from jax.experimental import pallas as pl
from jax.experimental.pallas import tpu as pltpu


def _conv_body(x_ref, w_ref, b_ref, o_ref, scr, *, TH, W, C_in, NT, IM):
    TS = TH * W
    HW = NT * TS
    FLAT = scr.shape[2]
    col = jax.lax.broadcasted_iota(jnp.int32, (C_in, HW), 1) % W
    b = b_ref[...]
    w = w_ref[...]
    for i in range(IM):
        flat = x_ref[i].astype(jnp.bfloat16).reshape(C_in, HW)
        zf = jnp.zeros((C_in, 2 * W), jnp.bfloat16)
        zb = jnp.zeros((C_in, 4 * W), jnp.bfloat16)
        for kw in range(3):
            scr[kw, :, 0:2 * W] = zf
            scr[kw, :, FLAT - 4 * W:FLAT] = zb
        scr[0, :, W + 1:W + 1 + HW] = jnp.where(col == W - 1, 0, flat)
        scr[1, :, W:W + HW] = flat
        scr[2, :, W - 1:W - 1 + HW] = jnp.where(col == 0, 0, flat)
        for t in range(NT):
            taps = []
            for kh in range(3):
                for kw in range(3):
                    off = t * TS + kh * W
                    taps.append(scr[kw, :, off:off + TS])
            xmat = jnp.concatenate(taps, axis=0)
            acc = jnp.dot(w, xmat, preferred_element_type=jnp.float32)
            o_ref[i, :, t * TS:(t + 1) * TS] = (acc + b).astype(o_ref.dtype)


def kernel(x, weight, bias):
    N, C_in, H, W = x.shape
    C_out = weight.shape[0]
    TH = 8
    HW = H * W
    TS = TH * W
    flat = (H // TH - 1) * TS + ((TS + 2 * W + 2 + 127) // 128) * 128


    w_mat = weight.transpose(0, 2, 3, 1).reshape(C_out, 9 * C_in)
    w_mat = w_mat.astype(jnp.bfloat16)
    b_col = bias.astype(jnp.float32).reshape(C_out, 1)

    IM = 4 if N % 4 == 0 else 1
    out = pl.pallas_call(
        functools.partial(_conv_body, TH=TH, W=W, C_in=C_in,
                          NT=H // TH, IM=IM),
        out_shape=jax.ShapeDtypeStruct((N, C_out, HW), jnp.bfloat16),
        grid=(N // IM,),
        in_specs=[
            pl.BlockSpec((IM, C_in, H, W), lambda n: (n, 0, 0, 0)),
            pl.BlockSpec((C_out, 9 * C_in), lambda n: (0, 0)),
            pl.BlockSpec((C_out, 1), lambda n: (0, 0)),
        ],
        out_specs=pl.BlockSpec((IM, C_out, HW), lambda n: (n, 0, 0)),
        scratch_shapes=[pltpu.VMEM((3, C_in, flat), jnp.bfloat16)],
        compiler_params=pltpu.CompilerParams(
            dimension_semantics=("arbitrary",)),
    )(x, w_mat, b_col)
    return out.reshape(N, C_out, H, W).astype(jnp.float32)
